# Initial kernel scaffold; baseline (speedup 1.0000x reference)
import jax
import jax.numpy as jnp
from jax.experimental import pallas as pl


def kernel(hidden_states, Wg, W1s, W2s, W1, W2):
    raise NotImplementedError("write your pallas kernel here")


# speedup vs baseline: 3.2349x; 3.2349x over previous
import functools

import jax
import jax.numpy as jnp
from jax.experimental import pallas as pl

H = 1024
I = 4096
E = 64
K = 8
F = 512
N = 2048

NEG = -1e30


def _router_body(x_ref, wg_ref, logits_ref, wfull_ref):
    x = x_ref[...]
    wg = wg_ref[...]
    logits = jax.lax.dot_general(
        x, wg, (((1,), (1,)), ((), ())), preferred_element_type=jnp.float32
    )
    iota_e = jax.lax.broadcasted_iota(jnp.int32, (N, E), 1)
    cur = logits
    vals = []
    sels = []
    for _ in range(K):
        m = jnp.max(cur, axis=1, keepdims=True)
        idx = jnp.min(jnp.where(cur == m, iota_e, E), axis=1, keepdims=True)
        vals.append(m)
        sels.append(idx)
        cur = jnp.where(iota_e == idx, NEG, cur)
    exps = [jnp.exp(v - vals[0]) for v in vals]
    denom = functools.reduce(jnp.add, exps)
    wfull = jnp.zeros((N, E), jnp.float32)
    for k in range(K):
        wfull = wfull + jnp.where(iota_e == sels[k], exps[k] / denom, 0.0)
    logits_ref[...] = logits
    wfull_ref[...] = wfull


def _shared_body(xb_ref, w1_ref, w2_ref, out_ref):
    i = pl.program_id(0)

    @pl.when(i == 0)
    def _():
        out_ref[...] = jnp.zeros_like(out_ref)

    xb = xb_ref[...]
    w1 = w1_ref[...].astype(jnp.bfloat16)
    h = jax.lax.dot_general(
        xb, w1, (((1,), (1,)), ((), ())), preferred_element_type=jnp.float32
    )
    h = h * jax.nn.sigmoid(h)
    w2 = w2_ref[...].astype(jnp.bfloat16)
    out_ref[...] += jax.lax.dot_general(
        h.astype(jnp.bfloat16), w2, (((1,), (1,)), ((), ())),
        preferred_element_type=jnp.float32,
    )


def _experts_body(xb_ref, w1_ref, w2_ref, wfull_ref, shared_ref, out_ref):
    e = pl.program_id(0)

    @pl.when(e == 0)
    def _():
        out_ref[...] = shared_ref[...]

    xb = xb_ref[...]
    w1 = w1_ref[0].astype(jnp.bfloat16)
    h = jax.lax.dot_general(
        xb, w1, (((1,), (1,)), ((), ())), preferred_element_type=jnp.float32
    )
    h = h * jax.nn.sigmoid(h)
    w2 = w2_ref[0].astype(jnp.bfloat16)
    o = jax.lax.dot_general(
        h.astype(jnp.bfloat16), w2, (((1,), (1,)), ((), ())),
        preferred_element_type=jnp.float32,
    )
    iota_e = jax.lax.broadcasted_iota(jnp.int32, (N, E), 1)
    we = jnp.sum(jnp.where(iota_e == e, wfull_ref[...], 0.0), axis=1,
                 keepdims=True)
    out_ref[...] += we * o


def kernel(hidden_states, Wg, W1s, W2s, W1, W2):
    b, s, h = hidden_states.shape
    x = hidden_states.reshape(-1, h)
    xb = x.astype(jnp.bfloat16)

    logits, wfull = pl.pallas_call(
        _router_body,
        out_shape=(
            jax.ShapeDtypeStruct((N, E), jnp.float32),
            jax.ShapeDtypeStruct((N, E), jnp.float32),
        ),
    )(x, Wg)

    IC = 512
    shared = pl.pallas_call(
        _shared_body,
        grid=(I // IC,),
        in_specs=[
            pl.BlockSpec((N, H), lambda i: (0, 0)),
            pl.BlockSpec((IC, H), lambda i: (i, 0)),
            pl.BlockSpec((H, IC), lambda i: (0, i)),
        ],
        out_specs=pl.BlockSpec((N, H), lambda i: (0, 0)),
        out_shape=jax.ShapeDtypeStruct((N, H), jnp.float32),
    )(xb, W1s, W2s)

    out = pl.pallas_call(
        _experts_body,
        grid=(E,),
        in_specs=[
            pl.BlockSpec((N, H), lambda e: (0, 0)),
            pl.BlockSpec((1, F, H), lambda e: (e, 0, 0)),
            pl.BlockSpec((1, H, F), lambda e: (e, 0, 0)),
            pl.BlockSpec((N, E), lambda e: (0, 0)),
            pl.BlockSpec((N, H), lambda e: (0, 0)),
        ],
        out_specs=pl.BlockSpec((N, H), lambda e: (0, 0)),
        out_shape=jax.ShapeDtypeStruct((N, H), jnp.float32),
    )(xb, W1, W2, wfull, shared)

    return (out.reshape(b, s, h), logits.reshape(b, s, E))
